# baseline (speedup 1.0000x reference)
import functools

import jax
import jax.numpy as jnp
from jax.experimental import pallas as pl
from jax.experimental.pallas import tpu as pltpu


def _hgc_kernel(nsteps, x_ref, dv_ref, h_ref, de_ref, out_ref, xn_ref):
    i = pl.program_id(0)

    @pl.when(i == 0)
    def _():
        xn_ref[...] = dv_ref[...] * x_ref[...]

    h = h_ref[...]
    y = jax.lax.dot_general(
        h, xn_ref[...], dimension_numbers=(((0,), (0,)), ((), ())),
        preferred_element_type=jnp.float32,
        precision=jax.lax.Precision.DEFAULT)
    y = y * de_ref[...]
    z = jax.lax.dot_general(
        h, y, dimension_numbers=(((1,), (0,)), ((), ())),
        preferred_element_type=jnp.float32,
        precision=jax.lax.Precision.DEFAULT)

    @pl.when(i == 0)
    def _():
        out_ref[...] = z

    @pl.when(i > 0)
    def _():
        out_ref[...] = out_ref[...] + z

    @pl.when(i == nsteps - 1)
    def _():
        out_ref[...] = out_ref[...] * dv_ref[...]


def kernel(X, H, Dv_inv_sqrt, De_inv):
    n, d = X.shape
    m = H.shape[1]
    bm = 256
    nsteps = m // bm
    dv = Dv_inv_sqrt.reshape(n, 1)
    de = De_inv.reshape(m, 1)
    return pl.pallas_call(
        functools.partial(_hgc_kernel, nsteps),
        grid=(nsteps,),
        in_specs=[
            pl.BlockSpec((n, d), lambda i: (0, 0)),
            pl.BlockSpec((n, 1), lambda i: (0, 0)),
            pl.BlockSpec((n, bm), lambda i: (0, i)),
            pl.BlockSpec((bm, 1), lambda i: (i, 0)),
        ],
        out_specs=pl.BlockSpec((n, d), lambda i: (0, 0)),
        out_shape=jax.ShapeDtypeStruct((n, d), X.dtype),
        scratch_shapes=[pltpu.VMEM((n, d), jnp.float32)],
    )(X, dv, H, de)


# speedup vs baseline: 1.4211x; 1.4211x over previous
import functools

import jax
import jax.numpy as jnp
from jax.experimental import pallas as pl
from jax.experimental.pallas import tpu as pltpu


def _hgc_kernel(nsteps, x_ref, dv_ref, h_ref, de_ref, out_ref, xn_ref):
    i = pl.program_id(0)

    @pl.when(i == 0)
    def _():
        xn_ref[...] = (dv_ref[...] * x_ref[...]).astype(jnp.bfloat16)

    h = h_ref[...].astype(jnp.bfloat16)
    y = jax.lax.dot_general(
        h, xn_ref[...], dimension_numbers=(((0,), (0,)), ((), ())),
        preferred_element_type=jnp.float32)
    y = (y * de_ref[...]).astype(jnp.bfloat16)
    z = jax.lax.dot_general(
        h, y, dimension_numbers=(((1,), (0,)), ((), ())),
        preferred_element_type=jnp.float32)

    @pl.when(i == 0)
    def _():
        out_ref[...] = z

    @pl.when(i > 0)
    def _():
        out_ref[...] = out_ref[...] + z

    @pl.when(i == nsteps - 1)
    def _():
        out_ref[...] = out_ref[...] * dv_ref[...]


def kernel(X, H, Dv_inv_sqrt, De_inv):
    n, d = X.shape
    m = H.shape[1]
    bm = 256
    nsteps = m // bm
    dv = Dv_inv_sqrt.reshape(n, 1)
    de = De_inv.reshape(m, 1)
    return pl.pallas_call(
        functools.partial(_hgc_kernel, nsteps),
        grid=(nsteps,),
        in_specs=[
            pl.BlockSpec((n, d), lambda i: (0, 0)),
            pl.BlockSpec((n, 1), lambda i: (0, 0)),
            pl.BlockSpec((n, bm), lambda i: (0, i)),
            pl.BlockSpec((bm, 1), lambda i: (i, 0)),
        ],
        out_specs=pl.BlockSpec((n, d), lambda i: (0, 0)),
        out_shape=jax.ShapeDtypeStruct((n, d), X.dtype),
        scratch_shapes=[pltpu.VMEM((n, d), jnp.bfloat16)],
    )(X, dv, H, de)
